# Initial kernel scaffold; baseline (speedup 1.0000x reference)
import jax
import jax.numpy as jnp
from jax.experimental import pallas as pl


def kernel(input, emb, W, b):
    raise NotImplementedError("write your pallas kernel here")


# speedup vs baseline: 1.3082x; 1.3082x over previous
import functools

import jax
import jax.numpy as jnp
from jax import lax
from jax.experimental import pallas as pl
from jax.experimental.pallas import tpu as pltpu
from jax.experimental.pallas import tpu_sc as plsc

_V = 4 * 10**6
_D = 5
_LANES = 128
_ROWS = (_V * _D) // (_D * _LANES)
_KDIM = _D * _LANES
_BR = 2048

_B, _L = 16384, 200
_N_IDX = _B * _L
_NC, _NS = 2, 16
_NW = _NC * _NS
_PER_W = _N_IDX // _NW
_CHUNK = 25600
_NCHUNK = _PER_W // _CHUNK


def _fuse_body(a_ref, m_ref, b_ref, o_ref):
    o_ref[...] = (
        jnp.dot(a_ref[...], m_ref[...], preferred_element_type=jnp.float32)
        + b_ref[0]
    )


def _fused_table(emb, W, b):
    a = emb.reshape(_ROWS, _KDIM)
    k = jnp.arange(_KDIM)
    m = jnp.where(
        (k[:, None] // _D) == jnp.arange(_LANES)[None, :],
        W[0][k % _D][:, None],
        0.0,
    ).astype(jnp.float32)
    grid = (_ROWS + _BR - 1) // _BR
    s2 = pl.pallas_call(
        _fuse_body,
        grid=(grid,),
        in_specs=[
            pl.BlockSpec((_BR, _KDIM), lambda i: (i, 0)),
            pl.BlockSpec((_KDIM, _LANES), lambda i: (0, 0)),
            pl.BlockSpec(memory_space=pltpu.SMEM),
        ],
        out_specs=pl.BlockSpec((_BR, _LANES), lambda i: (i, 0)),
        out_shape=jax.ShapeDtypeStruct((_ROWS, _LANES), jnp.float32),
    )(a, m, b)
    return s2.reshape(_V)


@functools.partial(
    pl.kernel,
    mesh=plsc.VectorSubcoreMesh(core_axis_name="c", subcore_axis_name="s"),
    out_type=jax.ShapeDtypeStruct((_N_IDX,), jnp.float32),
    scratch_types=[
        pltpu.VMEM((_CHUNK,), jnp.int32),
        pltpu.VMEM((_CHUNK,), jnp.float32),
        pltpu.SemaphoreType.DMA,
    ],
)
def _gather_scalars(s_hbm, idx_hbm, out_hbm, idx_v, val_v, sem):
    wid = lax.axis_index("s") * _NC + lax.axis_index("c")
    base = wid * _PER_W
    for i in range(_NCHUNK):
        off = base + i * _CHUNK
        pltpu.sync_copy(idx_hbm.at[pl.ds(off, _CHUNK)], idx_v)
        pltpu.async_copy(s_hbm.at[idx_v], val_v, sem).wait()
        pltpu.sync_copy(val_v, out_hbm.at[pl.ds(off, _CHUNK)])


def kernel(input, emb, W, b):
    idx = input.reshape(-1).astype(jnp.int32)
    s = _fused_table(emb, W, b)
    out = _gather_scalars(s, idx)
    return out.reshape(input.shape + (1,))
